# Initial kernel scaffold; baseline (speedup 1.0000x reference)
import jax
import jax.numpy as jnp
from jax.experimental import pallas as pl


def kernel(token_embeddings, pos_weight):
    raise NotImplementedError("write your pallas kernel here")


# speedup vs baseline: 1.6728x; 1.6728x over previous
import jax
import jax.numpy as jnp
from jax.experimental import pallas as pl


def _add_body(tok_ref, pos_ref, out_ref):
    out_ref[...] = tok_ref[...] + pos_ref[...]


def kernel(token_embeddings, pos_weight):
    B, S, D = token_embeddings.shape
    BS = 512
    grid = (S // BS, B)
    return pl.pallas_call(
        _add_body,
        grid=grid,
        in_specs=[
            pl.BlockSpec((1, BS, D), lambda s, b: (b, s, 0)),
            pl.BlockSpec((BS, D), lambda s, b: (s, 0)),
        ],
        out_specs=pl.BlockSpec((1, BS, D), lambda s, b: (b, s, 0)),
        out_shape=jax.ShapeDtypeStruct((B, S, D), token_embeddings.dtype),
    )(token_embeddings, pos_weight)


# speedup vs baseline: 1.7385x; 1.0393x over previous
import jax
import jax.numpy as jnp
from jax.experimental import pallas as pl


def _add_body(tok_ref, pos_ref, out_ref):
    out_ref[...] = tok_ref[...] + pos_ref[...]


def kernel(token_embeddings, pos_weight):
    B, S, D = token_embeddings.shape
    BS = 1024
    grid = (S // BS, B)
    return pl.pallas_call(
        _add_body,
        grid=grid,
        in_specs=[
            pl.BlockSpec((1, BS, D), lambda s, b: (b, s, 0)),
            pl.BlockSpec((BS, D), lambda s, b: (s, 0)),
        ],
        out_specs=pl.BlockSpec((1, BS, D), lambda s, b: (b, s, 0)),
        out_shape=jax.ShapeDtypeStruct((B, S, D), token_embeddings.dtype),
    )(token_embeddings, pos_weight)
